# Initial kernel scaffold; baseline (speedup 1.0000x reference)
import jax
import jax.numpy as jnp
from jax.experimental import pallas as pl


def kernel(z, embedding):
    raise NotImplementedError("write your pallas kernel here")


# speedup vs baseline: 1.5779x; 1.5779x over previous
import functools

import jax
import jax.numpy as jnp
from jax.experimental import pallas as pl
from jax.experimental.pallas import tpu as pltpu

_NUM_CODES = 1024
_EMBED_DIM = 64
_COMMIT = 0.25
_EPS = 1e-10
_BLOCK_T = 1024


def _vq_body(z_ref, e_ref, zq_ref, idx_ref, loss_ref, perp_ref, counts, acc):
    i = pl.program_id(0)
    k = pl.num_programs(0)
    t = z_ref.shape[0]
    n = t * k

    @pl.when(i == 0)
    def _init():
        counts[...] = jnp.zeros_like(counts)
        acc[...] = jnp.zeros_like(acc)

    z = z_ref[...]
    e = e_ref[...]
    zsq = jnp.sum(z * z, axis=1, keepdims=True)
    esq = jnp.sum(e * e, axis=1)
    prod = jax.lax.dot_general(
        z, e, (((1,), (1,)), ((), ())), preferred_element_type=jnp.float32)
    dist = zsq - 2.0 * prod + esq[None, :]
    mind = jnp.min(dist, axis=1, keepdims=True)
    iota = jax.lax.broadcasted_iota(jnp.int32, dist.shape, 1)
    idx = jnp.min(jnp.where(dist == mind, iota, _NUM_CODES), axis=1)
    onehot = (iota == idx[:, None]).astype(jnp.float32)
    zq = jax.lax.dot_general(
        onehot, e, (((1,), (0,)), ((), ())), preferred_element_type=jnp.float32)
    zq_ref[...] = zq
    idx_ref[...] = idx[:, None]
    counts[...] += jnp.sum(onehot, axis=0, keepdims=True)
    diff = z - zq
    acc[...] += jnp.sum(diff * diff, axis=(0, 1), keepdims=True)

    @pl.when(i == k - 1)
    def _fin():
        avg = counts[...] / n
        ent = jnp.sum(avg * jnp.log(avg + _EPS), axis=1, keepdims=True)
        perp_ref[...] = jnp.exp(-ent)
        loss_ref[...] = _COMMIT * acc[...] / (n * _EMBED_DIM)


def kernel(z, embedding):
    shape = z.shape
    zf = z.reshape(-1, _EMBED_DIM)
    n = zf.shape[0]
    t = _BLOCK_T
    k = n // t
    zq, idx, loss, perp = pl.pallas_call(
        _vq_body,
        grid=(k,),
        in_specs=[
            pl.BlockSpec((t, _EMBED_DIM), lambda i: (i, 0)),
            pl.BlockSpec((_NUM_CODES, _EMBED_DIM), lambda i: (0, 0)),
        ],
        out_specs=[
            pl.BlockSpec((t, _EMBED_DIM), lambda i: (i, 0)),
            pl.BlockSpec((t, 1), lambda i: (i, 0)),
            pl.BlockSpec((1, 1), lambda i: (0, 0)),
            pl.BlockSpec((1, 1), lambda i: (0, 0)),
        ],
        out_shape=[
            jax.ShapeDtypeStruct((n, _EMBED_DIM), jnp.float32),
            jax.ShapeDtypeStruct((n, 1), jnp.int32),
            jax.ShapeDtypeStruct((1, 1), jnp.float32),
            jax.ShapeDtypeStruct((1, 1), jnp.float32),
        ],
        scratch_shapes=[
            pltpu.VMEM((1, _NUM_CODES), jnp.float32),
            pltpu.VMEM((1, 1), jnp.float32),
        ],
    )(zf, embedding)
    return (
        zq.reshape(shape),
        idx[:, 0].reshape(shape[:-1]),
        loss[0, 0],
        perp[0, 0],
    )
